# baseline (speedup 1.0000x reference)
import functools

import jax
import jax.numpy as jnp
from jax import lax
from jax.experimental import pallas as pl
from jax.experimental.pallas import tpu as pltpu
from jax.experimental.pallas import tpu_sc as plsc

_N = 10000
_E = 320000
_D_IN = 128
_D_HID = 256
_D_OUT = 128

_NSUB = 16
_EPT = _E // _NSUB
_CHUNK = 80
_NCHUNK = _EPT // _CHUNK
_CP = 80
_NRC = _N // _CP
_ITER = (_NRC + _NSUB - 1) // _NSUB
_CNTW = 8


def _sc_agg(half_d, with_cnt, fused=False):
  mesh = plsc.VectorSubcoreMesh(core_axis_name="c", subcore_axis_name="s")
  out_type = [
      jax.ShapeDtypeStruct((_N, half_d), jnp.float32),
      jax.ShapeDtypeStruct((_N, half_d), jnp.float32),
  ]
  scratch = [
      pltpu.VMEM((_NCHUNK, _CHUNK), jnp.int32),
      pltpu.VMEM((_NCHUNK, _CHUNK), jnp.int32),
      pltpu.VMEM((_CHUNK, half_d), jnp.float32),
      pltpu.VMEM((_CHUNK, half_d), jnp.float32),
      pltpu.VMEM((_CHUNK, half_d), jnp.float32),
      pltpu.VMEM((_CHUNK, half_d), jnp.float32),
      pltpu.VMEM((_CP, half_d), jnp.float32),
      pltpu.SemaphoreType.DMA((4,)),
      pltpu.SemaphoreType.DMA((4,)),
      pltpu.SemaphoreType.DMA,
      pltpu.VMEM_SHARED((_N, half_d), jnp.float32),
  ]
  if with_cnt:
    out_type.append(jax.ShapeDtypeStruct((_N, _CNTW), jnp.float32))
    scratch += [
        pltpu.VMEM((_CHUNK, _CNTW), jnp.float32),
        pltpu.VMEM((_CP, _CNTW), jnp.float32),
        pltpu.VMEM_SHARED((_N, _CNTW), jnp.float32),
    ]
  if fused:
    scratch.append(pltpu.VMEM((_CP, half_d), jnp.float32))

  @functools.partial(
      pl.kernel, mesh=mesh, out_type=out_type, scratch_types=scratch,
      compiler_params=pltpu.CompilerParams(use_tc_tiling_on_sc=False))
  def agg(*refs):
    if with_cnt:
      (x0_hbm, x1_hbm, src_hbm, dst_hbm, z_hbm, zc_hbm, ones_hbm,
       s0_hbm, s1_hbm, cnt_hbm,
       srcv, dstv, rb0, rb1, rb2, rb3, tmp, gsem, ssem, csem,
       acc_sh, onesv, ctmp, cnt_sh) = refs
    elif fused:
      (x0_hbm, x1_hbm, src_hbm, dst_hbm, qc0_hbm, qc1_hbm, rm_hbm,
       s0_hbm, s1_hbm,
       srcv, dstv, rb0, rb1, rb2, rb3, tmp, gsem, ssem, csem,
       acc_sh, rtmp) = refs
    else:
      (x0_hbm, x1_hbm, src_hbm, dst_hbm, z_hbm,
       s0_hbm, s1_hbm,
       srcv, dstv, rb0, rb1, rb2, rb3, tmp, gsem, ssem, csem,
       acc_sh) = refs
    bufs = (rb0, rb1, rb2, rb3)
    cid = lax.axis_index("c")
    sid = lax.axis_index("s")

    pltpu.sync_copy(src_hbm.at[sid], srcv)
    pltpu.sync_copy(dst_hbm.at[sid], dstv)

    if fused:
      def _init_from(qcref):
        def _body(i, _):
          k = i * _NSUB + sid

          @pl.when(k < _NRC)
          def _():
            pltpu.sync_copy(qcref.at[pl.ds(k * _CP, _CP)], tmp)
            pltpu.sync_copy(tmp, acc_sh.at[pl.ds(k * _CP, _CP)])
          return 0

        lax.fori_loop(0, _ITER, _body, 0)

      @pl.when(cid == 0)
      def _():
        _init_from(qc0_hbm)

      @pl.when(cid == 1)
      def _():
        _init_from(qc1_hbm)
    else:
      pltpu.sync_copy(z_hbm, tmp)

      def _zero(i, _):
        k = i * _NSUB + sid

        @pl.when(k < _NRC)
        def _():
          pltpu.sync_copy(tmp, acc_sh.at[pl.ds(k * _CP, _CP)])
        return 0

      lax.fori_loop(0, _ITER, _zero, 0)

    if with_cnt:
      @pl.when(cid == 0)
      def _():
        pltpu.sync_copy(zc_hbm, ctmp)

        def _zc(i, _):
          k = i * _NSUB + sid

          @pl.when(k < _NRC)
          def _():
            pltpu.sync_copy(ctmp, cnt_sh.at[pl.ds(k * _CP, _CP)])
          return 0

        lax.fori_loop(0, _ITER, _zc, 0)
        pltpu.sync_copy(ones_hbm, onesv)

    plsc.subcore_barrier()

    def _run(xref, count):
      def _gstart(k, j):
        pltpu.async_copy(xref.at[srcv.at[k]], bufs[j], gsem.at[j])

      def _gwait(k, j):
        pltpu.make_async_copy(xref.at[srcv.at[k]], bufs[j],
                              gsem.at[j]).wait()

      def _sstart(k, j):
        pltpu.async_copy(bufs[j], acc_sh.at[dstv.at[k]], ssem.at[j],
                         add=True)

      def _swait(k, j):
        pltpu.make_async_copy(bufs[j], acc_sh.at[dstv.at[k]],
                              ssem.at[j]).wait()

      def _cstart(k):
        pltpu.async_copy(onesv, cnt_sh.at[dstv.at[k]], csem, add=True)

      def _cwait(k):
        pltpu.make_async_copy(onesv, cnt_sh.at[dstv.at[k]], csem).wait()

      def _step(k, j, prefetch):
        _gwait(k, j)
        _sstart(k, j)
        if count:
          _cstart(k)

          @pl.when(k >= 2)
          def _():
            _cwait(k - 2)
        if prefetch:
          j2 = (j + 2) % 4

          @pl.when(k >= 2)
          def _():
            _swait(k - 2, j2)

          _gstart(k + 2, j2)

      _gstart(0, 0)
      _gstart(1, 1)

      nquad = (_NCHUNK - 2) // 4

      def _quad(q, _):
        for j in range(4):
          _step(4 * q + j, j, True)
        return 0

      lax.fori_loop(0, nquad, _quad, 0)
      for k in range(4 * nquad, _NCHUNK):
        _step(k, k % 4, k + 2 < _NCHUNK)
      for k in range(_NCHUNK - 4, _NCHUNK):
        _swait(k, k % 4)
      if count:
        _cwait(_NCHUNK - 2)
        _cwait(_NCHUNK - 1)

    @pl.when(cid == 0)
    def _():
      _run(x0_hbm, with_cnt)

    @pl.when(cid == 1)
    def _():
      _run(x1_hbm, False)

    plsc.subcore_barrier()

    def _out(i, _):
      k = i * _NSUB + sid

      @pl.when(k < _NRC)
      def _():
        r0 = k * _CP
        pltpu.sync_copy(acc_sh.at[pl.ds(r0, _CP)], tmp)

        if fused:
          pltpu.sync_copy(rm_hbm.at[pl.ds(r0, _CP)], rtmp)
          for r in range(_CP):
            for c in range(half_d // 16):
              sl = pl.ds(c * 16, 16)
              tmp[r, sl] = tmp[r, sl] * rtmp[r, sl]

        @pl.when(cid == 0)
        def _():
          pltpu.sync_copy(tmp, s0_hbm.at[pl.ds(r0, _CP)])

        @pl.when(cid == 1)
        def _():
          pltpu.sync_copy(tmp, s1_hbm.at[pl.ds(r0, _CP)])

        if with_cnt:
          @pl.when(cid == 0)
          def _():
            pltpu.sync_copy(cnt_sh.at[pl.ds(r0, _CP)], ctmp)
            pltpu.sync_copy(ctmp, cnt_hbm.at[pl.ds(r0, _CP)])
      return 0

    lax.fori_loop(0, _ITER, _out, 0)

  return agg


_RB = 400


def _mm1_body(s0, s1, cnt, x, wl, b, wr, w2l, w2r, b2, p0, p1, qc0, qc1, rm):
  c = jnp.maximum(cnt[:, :1], 1.0)
  recip = 1.0 / c
  mean = jnp.concatenate([s0[:, :], s1[:, :]], axis=1) * recip
  t = (jnp.dot(mean, wl[:, :], preferred_element_type=jnp.float32)
       + jnp.dot(x[:, :], wr[:, :], preferred_element_type=jnp.float32)
       + b[:, :])
  t = jnp.maximum(t, 0.0)
  p = jnp.dot(t, w2l[:, :], preferred_element_type=jnp.float32)
  p0[:, :] = p[:, : _D_OUT // 2]
  p1[:, :] = p[:, _D_OUT // 2:]
  qc = (jnp.dot(t, w2r[:, :], preferred_element_type=jnp.float32)
        + b2[:, :]) * c
  qc0[:, :] = qc[:, : _D_OUT // 2]
  qc1[:, :] = qc[:, _D_OUT // 2:]
  rm[:, :] = jnp.broadcast_to(recip, (_RB, _D_OUT // 2))


def _row_spec(w):
  return pl.BlockSpec((_RB, w), lambda i: (i, 0))


def _full_spec(shape):
  return pl.BlockSpec(shape, lambda i: tuple(0 for _ in shape))


def kernel(x, edge_index, W1_l, b1_l, W1_r, W2_l, b2_l, W2_r):
  src = edge_index[0].reshape(_NSUB, _NCHUNK, _CHUNK)
  dst = edge_index[1].reshape(_NSUB, _NCHUNK, _CHUNK)
  x0 = x[:, : _D_IN // 2]
  x1 = x[:, _D_IN // 2:]
  z64 = jnp.zeros((_CP, _D_IN // 2), jnp.float32)
  zc = jnp.zeros((_CP, _CNTW), jnp.float32)
  ones = jnp.ones((_CHUNK, _CNTW), jnp.float32)

  s0, s1, cnt = _sc_agg(_D_IN // 2, True)(x0, x1, src, dst, z64, zc, ones)

  hw = _D_OUT // 2
  p0, p1, qc0, qc1, rm = pl.pallas_call(
      _mm1_body,
      grid=(_N // _RB,),
      in_specs=[
          _row_spec(_D_IN // 2), _row_spec(_D_IN // 2), _row_spec(_CNTW),
          _row_spec(_D_IN),
          _full_spec((_D_IN, _D_HID)), _full_spec((1, _D_HID)),
          _full_spec((_D_IN, _D_HID)), _full_spec((_D_HID, _D_OUT)),
          _full_spec((_D_HID, _D_OUT)), _full_spec((1, _D_OUT)),
      ],
      out_specs=[_row_spec(hw)] * 5,
      out_shape=[jax.ShapeDtypeStruct((_N, hw), jnp.float32)] * 5,
  )(s0, s1, cnt, x, W1_l, b1_l.reshape(1, -1), W1_r, W2_l, W2_r,
    b2_l.reshape(1, -1))

  o0, o1 = _sc_agg(hw, False, fused=True)(p0, p1, src, dst, qc0, qc1, rm)

  return jnp.concatenate([o0, o1], axis=1)


# speedup vs baseline: 1.0486x; 1.0486x over previous
import functools

import jax
import jax.numpy as jnp
from jax import lax
from jax.experimental import pallas as pl
from jax.experimental.pallas import tpu as pltpu
from jax.experimental.pallas import tpu_sc as plsc

_N = 10000
_E = 320000
_D_IN = 128
_D_HID = 256
_D_OUT = 128

_NSUB = 16
_EPT = _E // _NSUB
_CHUNK = 80
_NCHUNK = _EPT // _CHUNK
_CP = 80
_NRC = _N // _CP
_ITER = (_NRC + _NSUB - 1) // _NSUB
_CNTW = 8


def _sc_agg(half_d, with_cnt):
  mesh = plsc.VectorSubcoreMesh(core_axis_name="c", subcore_axis_name="s")
  out_type = [
      jax.ShapeDtypeStruct((_N, half_d), jnp.float32),
      jax.ShapeDtypeStruct((_N, half_d), jnp.float32),
  ]
  scratch = [
      pltpu.VMEM((_NCHUNK, _CHUNK), jnp.int32),
      pltpu.VMEM((_NCHUNK, _CHUNK), jnp.int32),
      pltpu.VMEM((_CHUNK, half_d), jnp.float32),
      pltpu.VMEM((_CHUNK, half_d), jnp.float32),
      pltpu.VMEM((_CHUNK, half_d), jnp.float32),
      pltpu.VMEM((_CHUNK, half_d), jnp.float32),
      pltpu.VMEM((_CP, half_d), jnp.float32),
      pltpu.SemaphoreType.DMA((4,)),
      pltpu.SemaphoreType.DMA((4,)),
      pltpu.SemaphoreType.DMA,
      pltpu.VMEM_SHARED((_N, half_d), jnp.float32),
  ]
  if with_cnt:
    out_type.append(jax.ShapeDtypeStruct((_N, _CNTW), jnp.float32))
    scratch += [
        pltpu.VMEM((_CHUNK, _CNTW), jnp.float32),
        pltpu.VMEM((_CP, _CNTW), jnp.float32),
        pltpu.VMEM_SHARED((_N, _CNTW), jnp.float32),
    ]

  @functools.partial(
      pl.kernel, mesh=mesh, out_type=out_type, scratch_types=scratch,
      compiler_params=pltpu.CompilerParams(use_tc_tiling_on_sc=False))
  def agg(*refs):
    if with_cnt:
      (x0_hbm, x1_hbm, src_hbm, dst_hbm, z_hbm, zc_hbm, ones_hbm,
       s0_hbm, s1_hbm, cnt_hbm,
       srcv, dstv, rb0, rb1, rb2, rb3, tmp, gsem, ssem, csem,
       acc_sh, onesv, ctmp, cnt_sh) = refs
    else:
      (x0_hbm, x1_hbm, src_hbm, dst_hbm, z_hbm,
       s0_hbm, s1_hbm,
       srcv, dstv, rb0, rb1, rb2, rb3, tmp, gsem, ssem, csem,
       acc_sh) = refs
    bufs = (rb0, rb1, rb2, rb3)
    cid = lax.axis_index("c")
    sid = lax.axis_index("s")

    pltpu.sync_copy(src_hbm.at[sid], srcv)
    pltpu.sync_copy(dst_hbm.at[sid], dstv)

    pltpu.sync_copy(z_hbm, tmp)

    def _zero(i, _):
      k = i * _NSUB + sid

      @pl.when(k < _NRC)
      def _():
        pltpu.sync_copy(tmp, acc_sh.at[pl.ds(k * _CP, _CP)])
      return 0

    lax.fori_loop(0, _ITER, _zero, 0)

    if with_cnt:
      @pl.when(cid == 0)
      def _():
        pltpu.sync_copy(zc_hbm, ctmp)

        def _zc(i, _):
          k = i * _NSUB + sid

          @pl.when(k < _NRC)
          def _():
            pltpu.sync_copy(ctmp, cnt_sh.at[pl.ds(k * _CP, _CP)])
          return 0

        lax.fori_loop(0, _ITER, _zc, 0)
        pltpu.sync_copy(ones_hbm, onesv)

    plsc.subcore_barrier()

    def _run(xref, count):
      def _gstart(k, j):
        pltpu.async_copy(xref.at[srcv.at[k]], bufs[j], gsem.at[j])

      def _gwait(k, j):
        pltpu.make_async_copy(xref.at[srcv.at[k]], bufs[j],
                              gsem.at[j]).wait()

      def _sstart(k, j):
        pltpu.async_copy(bufs[j], acc_sh.at[dstv.at[k]], ssem.at[j],
                         add=True)

      def _swait(k, j):
        pltpu.make_async_copy(bufs[j], acc_sh.at[dstv.at[k]],
                              ssem.at[j]).wait()

      def _cstart(k):
        pltpu.async_copy(onesv, cnt_sh.at[dstv.at[k]], csem, add=True)

      def _cwait(k):
        pltpu.make_async_copy(onesv, cnt_sh.at[dstv.at[k]], csem).wait()

      def _step(k, j, prefetch):
        _gwait(k, j)
        _sstart(k, j)
        if count:
          _cstart(k)

          @pl.when(k >= 2)
          def _():
            _cwait(k - 2)
        if prefetch:
          j2 = (j + 2) % 4

          @pl.when(k >= 2)
          def _():
            _swait(k - 2, j2)

          _gstart(k + 2, j2)

      _gstart(0, 0)
      _gstart(1, 1)

      nquad = (_NCHUNK - 2) // 4

      def _quad(q, _):
        for j in range(4):
          _step(4 * q + j, j, True)
        return 0

      lax.fori_loop(0, nquad, _quad, 0)
      for k in range(4 * nquad, _NCHUNK):
        _step(k, k % 4, k + 2 < _NCHUNK)
      for k in range(_NCHUNK - 4, _NCHUNK):
        _swait(k, k % 4)
      if count:
        _cwait(_NCHUNK - 2)
        _cwait(_NCHUNK - 1)

    @pl.when(cid == 0)
    def _():
      _run(x0_hbm, with_cnt)

    @pl.when(cid == 1)
    def _():
      _run(x1_hbm, False)

    plsc.subcore_barrier()

    def _out(i, _):
      k = i * _NSUB + sid

      @pl.when(k < _NRC)
      def _():
        r0 = k * _CP
        pltpu.sync_copy(acc_sh.at[pl.ds(r0, _CP)], tmp)

        @pl.when(cid == 0)
        def _():
          pltpu.sync_copy(tmp, s0_hbm.at[pl.ds(r0, _CP)])

        @pl.when(cid == 1)
        def _():
          pltpu.sync_copy(tmp, s1_hbm.at[pl.ds(r0, _CP)])

        if with_cnt:
          @pl.when(cid == 0)
          def _():
            pltpu.sync_copy(cnt_sh.at[pl.ds(r0, _CP)], ctmp)
            pltpu.sync_copy(ctmp, cnt_hbm.at[pl.ds(r0, _CP)])
      return 0

    lax.fori_loop(0, _ITER, _out, 0)

  return agg


_RB = 400


def _mm1_body(s0, s1, cnt, x, wl, b, wr, w2l, h, p0, p1):
  c = jnp.maximum(cnt[:, :1], 1.0)
  mean = jnp.concatenate([s0[:, :], s1[:, :]], axis=1) / c
  t = (jnp.dot(mean, wl[:, :], preferred_element_type=jnp.float32)
       + jnp.dot(x[:, :], wr[:, :], preferred_element_type=jnp.float32)
       + b[:, :])
  t = jnp.maximum(t, 0.0)
  h[:, :] = t
  p = jnp.dot(t, w2l[:, :], preferred_element_type=jnp.float32)
  p0[:, :] = p[:, : _D_OUT // 2]
  p1[:, :] = p[:, _D_OUT // 2:]


def _mm2_body(t0, t1, cnt, h, wr, b, out):
  c = jnp.maximum(cnt[:, :1], 1.0)
  out[:, :] = (jnp.concatenate([t0[:, :], t1[:, :]], axis=1) / c
               + jnp.dot(h[:, :], wr[:, :],
                         preferred_element_type=jnp.float32)
               + b[:, :])


def _row_spec(w):
  return pl.BlockSpec((_RB, w), lambda i: (i, 0))


def _full_spec(shape):
  return pl.BlockSpec(shape, lambda i: tuple(0 for _ in shape))


def kernel(x, edge_index, W1_l, b1_l, W1_r, W2_l, b2_l, W2_r):
  src = edge_index[0].reshape(_NSUB, _NCHUNK, _CHUNK)
  dst = edge_index[1].reshape(_NSUB, _NCHUNK, _CHUNK)
  x0 = x[:, : _D_IN // 2]
  x1 = x[:, _D_IN // 2:]
  z64 = jnp.zeros((_CP, _D_IN // 2), jnp.float32)
  zc = jnp.zeros((_CP, _CNTW), jnp.float32)
  ones = jnp.ones((_CHUNK, _CNTW), jnp.float32)

  s0, s1, cnt = _sc_agg(_D_IN // 2, True)(x0, x1, src, dst, z64, zc, ones)

  h, p0, p1 = pl.pallas_call(
      _mm1_body,
      grid=(_N // _RB,),
      in_specs=[
          _row_spec(_D_IN // 2), _row_spec(_D_IN // 2), _row_spec(_CNTW),
          _row_spec(_D_IN),
          _full_spec((_D_IN, _D_HID)), _full_spec((1, _D_HID)),
          _full_spec((_D_IN, _D_HID)), _full_spec((_D_HID, _D_OUT)),
      ],
      out_specs=[_row_spec(_D_HID),
                 _row_spec(_D_OUT // 2), _row_spec(_D_OUT // 2)],
      out_shape=[
          jax.ShapeDtypeStruct((_N, _D_HID), jnp.float32),
          jax.ShapeDtypeStruct((_N, _D_OUT // 2), jnp.float32),
          jax.ShapeDtypeStruct((_N, _D_OUT // 2), jnp.float32),
      ],
  )(s0, s1, cnt, x, W1_l, b1_l.reshape(1, -1), W1_r, W2_l)

  t0, t1 = _sc_agg(_D_OUT // 2, False)(p0, p1, src, dst, z64)

  out = pl.pallas_call(
      _mm2_body,
      grid=(_N // _RB,),
      in_specs=[
          _row_spec(_D_OUT // 2), _row_spec(_D_OUT // 2), _row_spec(_CNTW),
          _row_spec(_D_HID),
          _full_spec((_D_HID, _D_OUT)), _full_spec((1, _D_OUT)),
      ],
      out_specs=_row_spec(_D_OUT),
      out_shape=jax.ShapeDtypeStruct((_N, _D_OUT), jnp.float32),
  )(t0, t1, cnt, h, W2_r, b2_l.reshape(1, -1))

  return out
